# baseline (speedup 1.0000x reference)
import functools

import jax
import jax.numpy as jnp
from jax.experimental import pallas as pl
from jax.experimental.pallas import tpu as pltpu

_K = 8
_TT = 1024


def _body(tmpl_ref, surf_ref, gfeat_ref, pfhi_ref, pflo_ref,
          dW1a_ref, dW1b_ref, dW1c_ref, db1_ref, dW2_ref, db2_ref,
          dW3_ref, db3_ref,
          mW1a_ref, mW1b_ref, mW1c_ref, mb1_ref, mW2_ref, mb2_ref,
          mW3_ref, mb3_ref,
          disp_ref, mat_ref):
    f32 = jnp.float32
    tmpl = tmpl_ref[0]
    st = surf_ref[0]
    S = st.shape[1]

    tx, ty, tz = tmpl[:, 0:1], tmpl[:, 1:2], tmpl[:, 2:3]
    sx, sy, sz = st[0:1, :], st[1:2, :], st[2:3, :]
    t2 = tx * tx + ty * ty + tz * tz
    s2 = sx * sx + sy * sy + sz * sz
    dot = jax.lax.dot_general(
        tmpl.astype(jnp.bfloat16), st.astype(jnp.bfloat16),
        dimension_numbers=(((1,), (0,)), ((), ())),
        preferred_element_type=f32)
    d2 = (t2 + s2) - 2.0 * dot
    work = jnp.maximum(d2, 0.0)

    iota_f = jax.lax.broadcasted_iota(jnp.int32, work.shape, 1).astype(f32)
    big = float(S)
    for _ in range(_K):
        m = jnp.min(work, axis=1, keepdims=True)
        cand = jnp.where(work == m, iota_f, big)
        sel = jnp.min(cand, axis=1, keepdims=True)
        work = jnp.where(cand == sel, jnp.inf, work)
    msk = jnp.where(jnp.isinf(work), 1.0, 0.0).astype(jnp.bfloat16)

    dotb = functools.partial(jax.lax.dot_general,
                             dimension_numbers=(((1,), (0,)), ((), ())),
                             preferred_element_type=f32)

    def dot2(x, w):
        return dotb(x.astype(jnp.bfloat16), w.astype(jnp.bfloat16))

    pf_hi = pfhi_ref[0]
    pf_lo = pflo_ref[0]
    local = (dotb(msk, pf_hi) + dotb(msk, pf_lo)) * (1.0 / _K)
    g = gfeat_ref[0]

    h = dot2(tmpl, dW1a_ref[...]) + dot2(local, dW1b_ref[...])
    h = h + dot2(g, dW1c_ref[...]) + db1_ref[...]
    h = jnp.maximum(h, 0.0)
    h = jnp.maximum(dot2(h, dW2_ref[...]) + db2_ref[...], 0.0)
    disp = dot2(h, dW3_ref[...]) + db3_ref[...]
    disp_ref[...] = disp[None]

    m1 = dot2(tmpl, mW1a_ref[...]) + dot2(local, mW1b_ref[...])
    m1 = m1 + dot2(g, mW1c_ref[...]) + mb1_ref[...]
    m1 = jnp.maximum(m1, 0.0)
    m2 = jnp.maximum(dot2(m1, mW2_ref[...]) + mb2_ref[...], 0.0)
    m3 = dot2(m2, mW3_ref[...]) + mb3_ref[...]
    mat_ref[...] = jax.nn.sigmoid(m3)[None]


def _run(template, surf_t, global_feat, pf_hi, pf_lo, weights, interpret=False):
    B, T, _ = template.shape
    S = surf_t.shape[2]
    D = pf_hi.shape[2]
    NT = T // _TT
    (dW1a, dW1b, dW1c, db1, dW2, db2, dW3, db3,
     mW1a, mW1b, mW1c, mb1, mW2, mb2, mW3, mb3) = weights

    def full(a):
        return pl.BlockSpec(a.shape, lambda b, t: (0,) * a.ndim)

    grid = (B, NT)
    in_specs = [
        pl.BlockSpec((1, _TT, 3), lambda b, t: (b, t, 0)),
        pl.BlockSpec((1, 3, S), lambda b, t: (b, 0, 0)),
        pl.BlockSpec((1, 1, global_feat.shape[2]), lambda b, t: (b, 0, 0)),
        pl.BlockSpec((1, S, D), lambda b, t: (b, 0, 0)),
        pl.BlockSpec((1, S, D), lambda b, t: (b, 0, 0)),
    ] + [full(w) for w in weights]
    out_specs = [
        pl.BlockSpec((1, _TT, 3), lambda b, t: (b, t, 0)),
        pl.BlockSpec((1, _TT, 1), lambda b, t: (b, t, 0)),
    ]
    out_shape = [
        jax.ShapeDtypeStruct((B, T, 3), jnp.float32),
        jax.ShapeDtypeStruct((B, T, 1), jnp.float32),
    ]
    disp, mat = pl.pallas_call(
        _body, grid=grid, in_specs=in_specs, out_specs=out_specs,
        out_shape=out_shape, interpret=interpret,
        compiler_params=pltpu.CompilerParams(
            dimension_semantics=("parallel", "arbitrary")),
    )(template, surf_t, global_feat, pf_hi, pf_lo, *weights)
    return disp, mat[..., 0]


def kernel(template, surf_xyz, global_feat, point_feat,
           dW1, db1, dW2, db2, dW3, db3,
           mW1, mb1, mW2, mb2, mW3, mb3):
    surf_t = jnp.transpose(surf_xyz, (0, 2, 1))
    global_feat = global_feat[:, None, :]
    pf_hi = point_feat.astype(jnp.bfloat16)
    pf_lo = (point_feat - pf_hi.astype(jnp.float32)).astype(jnp.bfloat16)
    D = point_feat.shape[2]
    weights = (
        dW1[0:3], dW1[3:3 + D], dW1[3 + D:], db1[None, :],
        dW2, db2[None, :], dW3, db3[None, :],
        mW1[0:3], mW1[3:3 + D], mW1[3 + D:], mb1[None, :],
        mW2, mb2[None, :], mW3, mb3[None, :],
    )
    return _run(template, surf_t, global_feat, pf_hi, pf_lo, weights)


# speedup vs baseline: 1.1579x; 1.1579x over previous
import functools

import jax
import jax.numpy as jnp
from jax.experimental import pallas as pl
from jax.experimental.pallas import tpu as pltpu

_K = 8
_TT = 256


def _body(tmpl_ref, surf_ref, gfeat_ref, pfhi_ref, pflo_ref,
          dW1a_ref, dW1b_ref, dW1c_ref, db1_ref, dW2_ref, db2_ref,
          dW3_ref, db3_ref,
          mW1a_ref, mW1b_ref, mW1c_ref, mb1_ref, mW2_ref, mb2_ref,
          mW3_ref, mb3_ref,
          disp_ref, mat_ref):
    f32 = jnp.float32
    tmpl = tmpl_ref[0]
    st = surf_ref[0]
    S = st.shape[1]

    tx, ty, tz = tmpl[:, 0:1], tmpl[:, 1:2], tmpl[:, 2:3]
    sx, sy, sz = st[0:1, :], st[1:2, :], st[2:3, :]
    t2 = tx * tx + ty * ty + tz * tz
    s2 = sx * sx + sy * sy + sz * sz
    dot = jax.lax.dot_general(
        tmpl.astype(jnp.bfloat16), st.astype(jnp.bfloat16),
        dimension_numbers=(((1,), (0,)), ((), ())),
        preferred_element_type=f32)
    d2 = (t2 + s2) - 2.0 * dot
    work = jnp.maximum(d2, 0.0)

    iota_f = jax.lax.broadcasted_iota(jnp.int32, work.shape, 1).astype(f32)
    big = float(S)
    for _ in range(_K):
        m = jnp.min(work, axis=1, keepdims=True)
        cand = jnp.where(work == m, iota_f, big)
        sel = jnp.min(cand, axis=1, keepdims=True)
        work = jnp.where(cand == sel, jnp.inf, work)
    msk = jnp.where(jnp.isinf(work), 1.0, 0.0).astype(jnp.bfloat16)

    dotb = functools.partial(jax.lax.dot_general,
                             dimension_numbers=(((1,), (0,)), ((), ())),
                             preferred_element_type=f32)

    def dot2(x, w):
        return dotb(x.astype(jnp.bfloat16), w.astype(jnp.bfloat16))

    pf_hi = pfhi_ref[0]
    pf_lo = pflo_ref[0]
    local = (dotb(msk, pf_hi) + dotb(msk, pf_lo)) * (1.0 / _K)
    g = gfeat_ref[0]

    h = dot2(tmpl, dW1a_ref[...]) + dot2(local, dW1b_ref[...])
    h = h + dot2(g, dW1c_ref[...]) + db1_ref[...]
    h = jnp.maximum(h, 0.0)
    h = jnp.maximum(dot2(h, dW2_ref[...]) + db2_ref[...], 0.0)
    disp = dot2(h, dW3_ref[...]) + db3_ref[...]
    disp_ref[...] = disp[None]

    m1 = dot2(tmpl, mW1a_ref[...]) + dot2(local, mW1b_ref[...])
    m1 = m1 + dot2(g, mW1c_ref[...]) + mb1_ref[...]
    m1 = jnp.maximum(m1, 0.0)
    m2 = jnp.maximum(dot2(m1, mW2_ref[...]) + mb2_ref[...], 0.0)
    m3 = dot2(m2, mW3_ref[...]) + mb3_ref[...]
    mat_ref[...] = jax.nn.sigmoid(m3)[None]


def _run(template, surf_t, global_feat, pf_hi, pf_lo, weights, interpret=False):
    B, T, _ = template.shape
    S = surf_t.shape[2]
    D = pf_hi.shape[2]
    NT = T // _TT
    (dW1a, dW1b, dW1c, db1, dW2, db2, dW3, db3,
     mW1a, mW1b, mW1c, mb1, mW2, mb2, mW3, mb3) = weights

    def full(a):
        return pl.BlockSpec(a.shape, lambda b, t: (0,) * a.ndim)

    grid = (B, NT)
    in_specs = [
        pl.BlockSpec((1, _TT, 3), lambda b, t: (b, t, 0)),
        pl.BlockSpec((1, 3, S), lambda b, t: (b, 0, 0)),
        pl.BlockSpec((1, 1, global_feat.shape[2]), lambda b, t: (b, 0, 0)),
        pl.BlockSpec((1, S, D), lambda b, t: (b, 0, 0)),
        pl.BlockSpec((1, S, D), lambda b, t: (b, 0, 0)),
    ] + [full(w) for w in weights]
    out_specs = [
        pl.BlockSpec((1, _TT, 3), lambda b, t: (b, t, 0)),
        pl.BlockSpec((1, _TT, 1), lambda b, t: (b, t, 0)),
    ]
    out_shape = [
        jax.ShapeDtypeStruct((B, T, 3), jnp.float32),
        jax.ShapeDtypeStruct((B, T, 1), jnp.float32),
    ]
    disp, mat = pl.pallas_call(
        _body, grid=grid, in_specs=in_specs, out_specs=out_specs,
        out_shape=out_shape, interpret=interpret,
        compiler_params=pltpu.CompilerParams(
            dimension_semantics=("parallel", "arbitrary")),
    )(template, surf_t, global_feat, pf_hi, pf_lo, *weights)
    return disp, mat[..., 0]


def kernel(template, surf_xyz, global_feat, point_feat,
           dW1, db1, dW2, db2, dW3, db3,
           mW1, mb1, mW2, mb2, mW3, mb3):
    surf_t = jnp.transpose(surf_xyz, (0, 2, 1))
    global_feat = global_feat[:, None, :]
    pf_hi = point_feat.astype(jnp.bfloat16)
    pf_lo = (point_feat - pf_hi.astype(jnp.float32)).astype(jnp.bfloat16)
    D = point_feat.shape[2]
    weights = (
        dW1[0:3], dW1[3:3 + D], dW1[3 + D:], db1[None, :],
        dW2, db2[None, :], dW3, db3[None, :],
        mW1[0:3], mW1[3:3 + D], mW1[3 + D:], mb1[None, :],
        mW2, mb2[None, :], mW3, mb3[None, :],
    )
    return _run(template, surf_t, global_feat, pf_hi, pf_lo, weights)


# speedup vs baseline: 1.2318x; 1.0638x over previous
import functools

import jax
import jax.numpy as jnp
from jax.experimental import pallas as pl
from jax.experimental.pallas import tpu as pltpu

_K = 8
_TT = 512


def _body(tmpl_ref, surf_ref, gfeat_ref, pfhi_ref, pflo_ref,
          dW1a_ref, dW1b_ref, dW1c_ref, db1_ref, dW2_ref, db2_ref,
          dW3_ref, db3_ref,
          mW1a_ref, mW1b_ref, mW1c_ref, mb1_ref, mW2_ref, mb2_ref,
          mW3_ref, mb3_ref,
          disp_ref, mat_ref):
    f32 = jnp.float32
    tmpl = tmpl_ref[0]
    st = surf_ref[0]
    S = st.shape[1]

    tx, ty, tz = tmpl[:, 0:1], tmpl[:, 1:2], tmpl[:, 2:3]
    sx, sy, sz = st[0:1, :], st[1:2, :], st[2:3, :]
    t2 = tx * tx + ty * ty + tz * tz
    s2 = sx * sx + sy * sy + sz * sz
    dot = jax.lax.dot_general(
        tmpl.astype(jnp.bfloat16), st.astype(jnp.bfloat16),
        dimension_numbers=(((1,), (0,)), ((), ())),
        preferred_element_type=f32)
    d2 = (t2 + s2) - 2.0 * dot
    work = jnp.maximum(d2, 0.0)

    iota_f = jax.lax.broadcasted_iota(jnp.int32, work.shape, 1).astype(f32)
    big = float(S)
    for _ in range(_K):
        m = jnp.min(work, axis=1, keepdims=True)
        cand = jnp.where(work == m, iota_f, big)
        sel = jnp.min(cand, axis=1, keepdims=True)
        work = jnp.where(cand == sel, jnp.inf, work)
    msk = jnp.where(jnp.isinf(work), 1.0, 0.0).astype(jnp.bfloat16)

    dotb = functools.partial(jax.lax.dot_general,
                             dimension_numbers=(((1,), (0,)), ((), ())),
                             preferred_element_type=f32)

    def dot2(x, w):
        return dotb(x.astype(jnp.bfloat16), w.astype(jnp.bfloat16))

    pf_hi = pfhi_ref[0]
    pf_lo = pflo_ref[0]
    local = (dotb(msk, pf_hi) + dotb(msk, pf_lo)) * (1.0 / _K)
    g = gfeat_ref[0]

    h = dot2(tmpl, dW1a_ref[...]) + dot2(local, dW1b_ref[...])
    h = h + dot2(g, dW1c_ref[...]) + db1_ref[...]
    h = jnp.maximum(h, 0.0)
    h = jnp.maximum(dot2(h, dW2_ref[...]) + db2_ref[...], 0.0)
    disp = dot2(h, dW3_ref[...]) + db3_ref[...]
    disp_ref[...] = disp[None]

    m1 = dot2(tmpl, mW1a_ref[...]) + dot2(local, mW1b_ref[...])
    m1 = m1 + dot2(g, mW1c_ref[...]) + mb1_ref[...]
    m1 = jnp.maximum(m1, 0.0)
    m2 = jnp.maximum(dot2(m1, mW2_ref[...]) + mb2_ref[...], 0.0)
    m3 = dot2(m2, mW3_ref[...]) + mb3_ref[...]
    mat_ref[...] = jax.nn.sigmoid(m3)[None]


def _run(template, surf_t, global_feat, pf_hi, pf_lo, weights, interpret=False):
    B, T, _ = template.shape
    S = surf_t.shape[2]
    D = pf_hi.shape[2]
    NT = T // _TT
    (dW1a, dW1b, dW1c, db1, dW2, db2, dW3, db3,
     mW1a, mW1b, mW1c, mb1, mW2, mb2, mW3, mb3) = weights

    def full(a):
        return pl.BlockSpec(a.shape, lambda b, t: (0,) * a.ndim)

    grid = (B, NT)
    in_specs = [
        pl.BlockSpec((1, _TT, 3), lambda b, t: (b, t, 0)),
        pl.BlockSpec((1, 3, S), lambda b, t: (b, 0, 0)),
        pl.BlockSpec((1, 1, global_feat.shape[2]), lambda b, t: (b, 0, 0)),
        pl.BlockSpec((1, S, D), lambda b, t: (b, 0, 0)),
        pl.BlockSpec((1, S, D), lambda b, t: (b, 0, 0)),
    ] + [full(w) for w in weights]
    out_specs = [
        pl.BlockSpec((1, _TT, 3), lambda b, t: (b, t, 0)),
        pl.BlockSpec((1, _TT, 1), lambda b, t: (b, t, 0)),
    ]
    out_shape = [
        jax.ShapeDtypeStruct((B, T, 3), jnp.float32),
        jax.ShapeDtypeStruct((B, T, 1), jnp.float32),
    ]
    disp, mat = pl.pallas_call(
        _body, grid=grid, in_specs=in_specs, out_specs=out_specs,
        out_shape=out_shape, interpret=interpret,
        compiler_params=pltpu.CompilerParams(
            dimension_semantics=("parallel", "arbitrary")),
    )(template, surf_t, global_feat, pf_hi, pf_lo, *weights)
    return disp, mat[..., 0]


def kernel(template, surf_xyz, global_feat, point_feat,
           dW1, db1, dW2, db2, dW3, db3,
           mW1, mb1, mW2, mb2, mW3, mb3):
    surf_t = jnp.transpose(surf_xyz, (0, 2, 1))
    global_feat = global_feat[:, None, :]
    pf_hi = point_feat.astype(jnp.bfloat16)
    pf_lo = (point_feat - pf_hi.astype(jnp.float32)).astype(jnp.bfloat16)
    D = point_feat.shape[2]
    weights = (
        dW1[0:3], dW1[3:3 + D], dW1[3 + D:], db1[None, :],
        dW2, db2[None, :], dW3, db3[None, :],
        mW1[0:3], mW1[3:3 + D], mW1[3 + D:], mb1[None, :],
        mW2, mb2[None, :], mW3, mb3[None, :],
    )
    return _run(template, surf_t, global_feat, pf_hi, pf_lo, weights)
